# baseline (speedup 1.0000x reference)
import jax
import jax.numpy as jnp
from jax import lax
from jax.experimental import pallas as pl
from jax.experimental.pallas import tpu as pltpu
from jax.experimental.pallas import tpu_sc as plsc

_CT_COLS = 128
_IND_CHUNK = 128

_NW = 32
_SLOTS_PER_W = 4
_PARTS = 4
_PART_ROWS = 16
_NBUF = 3
_IND_WORKERS = 25
_IND_PER_W = 4000


def _tc_body(s_ref, ct_ref, ind_ref, t_ref, ct_out, chunk_smem, sem):
    idx = s_ref[0]

    base = pl.multiple_of((idx // _IND_CHUNK) * _IND_CHUNK, _IND_CHUNK)
    cur_cp = pltpu.make_async_copy(
        ind_ref.at[pl.ds(base, _IND_CHUNK)], chunk_smem, sem)
    cur_cp.start()
    cur_cp.wait()
    cursor = chunk_smem[idx - base]

    cc = idx - (idx // _CT_COLS) * _CT_COLS
    row_i = jax.lax.broadcasted_iota(jnp.int32, ct_ref.shape, 0)
    col_i = jax.lax.broadcasted_iota(jnp.int32, ct_ref.shape, 1)
    ct_out[...] = jnp.where((row_i == cursor) & (col_i == cc),
                            t_ref[0], ct_ref[...])


def _sc_body(idx16_ref, si16_ref, ind_ref, saved_ref, new_ref,
             ind_out, saved_out, bufs, indbuf, sbuf, in_sems, out_sems):
    wid = lax.axis_index("s") * 2 + lax.axis_index("c")

    pltpu.sync_copy(idx16_ref, sbuf)
    idx = jnp.max(sbuf[...])
    pltpu.sync_copy(si16_ref, sbuf)
    si = jnp.max(sbuf[...])

    n_chunks = _SLOTS_PER_W * _PARTS

    def chunk_coords(j):
        slot = wid * _SLOTS_PER_W + (j // _PARTS)
        part = (j % _PARTS) * _PART_ROWS
        return slot, part

    def start_in(j):
        slot, part = chunk_coords(j)
        buf = bufs.at[j % _NBUF]
        sem = in_sems.at[j % _NBUF]

        @pl.when(slot == si)
        def _():
            pltpu.make_async_copy(
                new_ref.at[slot, pl.ds(part, _PART_ROWS), :], buf, sem
            ).start()

        @pl.when(slot != si)
        def _():
            pltpu.make_async_copy(
                saved_ref.at[slot, pl.ds(part, _PART_ROWS), :], buf, sem
            ).start()

    def wait_in(j):
        slot, part = chunk_coords(j)
        pltpu.make_async_copy(
            saved_ref.at[slot, pl.ds(part, _PART_ROWS), :],
            bufs.at[j % _NBUF], in_sems.at[j % _NBUF]).wait()

    def out_cp(j):
        slot, part = chunk_coords(j)
        return pltpu.make_async_copy(
            bufs.at[j % _NBUF],
            saved_out.at[slot, pl.ds(part, _PART_ROWS), :],
            out_sems.at[j % _NBUF])

    for j in range(_NBUF):
        start_in(j)
    for k in range(n_chunks):
        wait_in(k)
        out_cp(k).start()
        if k >= 1:
            out_cp(k - 1).wait()
            j = k - 1 + _NBUF
            if j < n_chunks:
                start_in(j)
    out_cp(n_chunks - 1).wait()

    @pl.when(wid < _IND_WORKERS)
    def _():
        base = wid * _IND_PER_W
        pltpu.sync_copy(ind_ref.at[pl.ds(base, _IND_PER_W)], indbuf)

        @pl.when(wid == idx // _IND_PER_W)
        def _():
            g = pl.multiple_of((idx // 16) * 16 - base, 8)
            v = indbuf[pl.ds(g, 16)]
            lane = idx - (idx // 16) * 16
            indbuf[pl.ds(g, 16)] = jnp.where(
                lax.iota(jnp.int32, 16) == lane, v + 1, v)

        pltpu.sync_copy(indbuf, ind_out.at[pl.ds(base, _IND_PER_W)])


def kernel(clicktimes, indices, idx, t, saved, new, save_index):
    idx32 = jnp.asarray(idx, jnp.int32)
    si32 = jnp.asarray(save_index, jnp.int32)
    t_arr = jnp.asarray(t, jnp.float32).reshape(1)
    idx16 = jnp.full((16,), idx32, jnp.int32)
    si16 = jnp.full((16,), si32, jnp.int32)

    ct_t = clicktimes.T
    saved_t = saved.transpose(0, 2, 1)
    new_t = new.transpose(0, 2, 1)

    mesh = plsc.VectorSubcoreMesh(core_axis_name="c", subcore_axis_name="s")
    sc_fn = pl.kernel(
        _sc_body,
        out_type=[
            jax.ShapeDtypeStruct(indices.shape, indices.dtype),
            jax.ShapeDtypeStruct(saved_t.shape, saved_t.dtype),
        ],
        mesh=mesh,
        scratch_types=[
            pltpu.VMEM((_NBUF, _PART_ROWS, saved_t.shape[2]), saved_t.dtype),
            pltpu.VMEM((_IND_PER_W,), indices.dtype),
            pltpu.VMEM((16,), jnp.int32),
            pltpu.SemaphoreType.DMA((_NBUF,)),
            pltpu.SemaphoreType.DMA((_NBUF,)),
        ],
        compiler_params=pltpu.CompilerParams(needs_layout_passes=False),
    )
    ind_out, saved_out_t = sc_fn(idx16, si16, indices, saved_t, new_t)

    n_clicks = ct_t.shape[0]
    grid_spec = pltpu.PrefetchScalarGridSpec(
        num_scalar_prefetch=1,
        grid=(1,),
        in_specs=[
            pl.BlockSpec((n_clicks, _CT_COLS),
                         lambda i, s: (0, s[0] // _CT_COLS)),
            pl.BlockSpec(memory_space=pltpu.HBM),
            pl.BlockSpec(memory_space=pltpu.SMEM),
        ],
        out_specs=[
            pl.BlockSpec((n_clicks, _CT_COLS),
                         lambda i, s: (0, s[0] // _CT_COLS)),
        ],
        scratch_shapes=[
            pltpu.SMEM((_IND_CHUNK,), indices.dtype),
            pltpu.SemaphoreType.DMA,
        ],
    )
    (ct_out_t,) = pl.pallas_call(
        _tc_body,
        grid_spec=grid_spec,
        out_shape=[jax.ShapeDtypeStruct(ct_t.shape, ct_t.dtype)],
        input_output_aliases={1: 0},
    )(jnp.stack([idx32]), ct_t, indices, t_arr)

    return (ct_out_t.T, ind_out, saved_out_t.transpose(0, 2, 1),
            save_index + 1)
